# baseline (speedup 1.0000x reference)
import functools

import jax
import jax.numpy as jnp
from jax import lax
from jax.experimental import pallas as pl
from jax.experimental.pallas import tpu as pltpu
from jax.experimental.pallas import tpu_sc as plsc


N = 10000
H = 9
NW = 32
CHUNK = 64
EPAD = N * 32 + 128
NPAD = 10112
RS = 80
NPAD2 = 128 * RS


def _lane_scalar(v, lane):
    lanes = lax.iota(jnp.int32, 16)
    t = jnp.where(lanes == lane, v, jnp.zeros((16,), v.dtype))
    for s in (8, 4, 2, 1):
        t = t + t[(lanes + s) % 16]
    return t[0]


def _feast_edge_body(hoc, hocp, oc, chunk, t1_hbm, xud_hbm, src_hbm, seg_hbm,
                     c_hbm, bnd_hbm, out_hbm, src_v, seg_v, t1_v, xud_v,
                     c_v, bnd_v, acc_v, sem):
    cid = lax.axis_index("c")
    sid = lax.axis_index("s")
    wid = sid * 2 + cid

    def zrow(r, carry):
        for cb in range(8):
            acc_v[r, pl.ds(cb * 16, 16)] = jnp.zeros((16,), jnp.float32)
        return carry

    lax.fori_loop(0, 4 * RS + 8, zrow, 0)
    pltpu.sync_copy(c_hbm, c_v)
    pltpu.sync_copy(bnd_hbm, bnd_v)
    lanes = lax.iota(jnp.int32, 16)
    neg = jnp.where(lanes < H, 0.0, -1e30)
    one0 = jnp.where(lanes < 1, 1.0, 0.0)

    def bnd_at(idx):
        base = pl.multiple_of((idx // 16) * 16, 16)
        return _lane_scalar(bnd_v[pl.ds(base, 16)], idx - base)

    for k in range(4):
        rid = wid + 32 * k
        lo = bnd_at(rid)
        hi = bnd_at(rid + 1)
        off = rid * RS - k * RS
        albase = pl.multiple_of(lo - jnp.mod(lo, 8), 8)
        ntr = jnp.maximum((hi - albase + chunk - 1) // chunk, 0)

        def chunk_body(ch, carry, albase=albase, lo=lo, hi=hi, off=off):
            base = pl.multiple_of(albase + ch * chunk, 8)
            pltpu.sync_copy(src_hbm.at[pl.ds(base, chunk)], src_v)
            pltpu.sync_copy(seg_hbm.at[pl.ds(base, chunk)], seg_v)
            pltpu.async_copy(xud_hbm.at[seg_v], xud_v, sem).wait()
            pltpu.async_copy(t1_hbm.at[src_v], t1_v, sem).wait()

            def edge_body(i, c2):
                sgv = seg_v[pl.ds(pl.multiple_of((i // 16) * 16, 16), 16)]
                sseg = _lane_scalar(sgv, jnp.mod(i, 16))
                j = base + i
                valid = (j >= lo) & (j < hi)
                lrow = jnp.where(valid, sseg - off, 4 * RS)
                z = (t1_v[i, pl.ds(hoc, 16)] - xud_v[i, pl.ds(0, 16)]
                     + c_v[...] + neg)
                m = z
                for s in (8, 4, 2, 1):
                    m = jnp.maximum(m, m[(lanes + s) % 16])
                e = jnp.exp(z - m)
                t = e
                for s in (8, 4, 2, 1):
                    t = t + t[(lanes + s) % 16]
                q = e / t
                accs = [jnp.zeros((16,), jnp.float32) for _ in range(oc // 16)]
                for h in range(H):
                    qh = q[h]
                    for cb in range(oc // 16):
                        accs[cb] = (accs[cb]
                                    + qh * t1_v[i, pl.ds(h * oc + cb * 16, 16)])
                for cb in range(oc // 16):
                    plsc.addupdate(acc_v.at[lrow, pl.ds(cb * 16, 16)], accs[cb])
                if oc < 128:
                    plsc.addupdate(acc_v.at[lrow, pl.ds(oc, 16)], one0)
                return c2

            lax.fori_loop(0, chunk, edge_body, 0)
            return carry

        lax.fori_loop(0, ntr, chunk_body, 0)

    for k in range(4):
        ofs = pl.multiple_of((wid + 32 * k) * RS, 8)
        pltpu.sync_copy(acc_v.at[pl.ds(k * RS, RS)], out_hbm.at[pl.ds(ofs, RS)])


@functools.lru_cache(maxsize=None)
def _feast_edge_kernel(hoc, hocp, oc):
    chunk = 32 if hocp > 640 else 64
    mesh = plsc.VectorSubcoreMesh(core_axis_name="c", subcore_axis_name="s")
    return pl.kernel(
        functools.partial(_feast_edge_body, hoc, hocp, oc, chunk),
        mesh=mesh,
        out_type=jax.ShapeDtypeStruct((NPAD2, 128), jnp.float32),
        scratch_types=[
            pltpu.VMEM((chunk,), jnp.int32),
            pltpu.VMEM((chunk,), jnp.int32),
            pltpu.VMEM((chunk, hocp), jnp.float32),
            pltpu.VMEM((chunk, 128), jnp.float32),
            pltpu.VMEM((16,), jnp.float32),
            pltpu.VMEM((144,), jnp.int32),
            pltpu.VMEM((4 * RS + 8, 128), jnp.float32),
            pltpu.SemaphoreType.DMA,
        ],
    )


def _round_up(v, m):
    return (v + m - 1) // m * m


def _mm_body(x_ref, w_ref, o_ref):
    o_ref[...] = jnp.dot(x_ref[...], w_ref[...],
                         preferred_element_type=jnp.float32)


def _pallas_matmul(x, w):
    n, ic = x.shape
    k = w.shape[1]
    BN = 1024
    npad = _round_up(n, BN)
    icp = _round_up(ic, 128)
    kp = _round_up(k, 128)
    xp = jnp.zeros((npad, icp), jnp.float32).at[:n, :ic].set(x)
    wp = jnp.zeros((icp, kp), jnp.float32).at[:ic, :k].set(w)
    out = pl.pallas_call(
        _mm_body,
        grid=(npad // BN,),
        in_specs=[pl.BlockSpec((BN, icp), lambda i: (i, 0)),
                  pl.BlockSpec((icp, kp), lambda i: (0, 0))],
        out_specs=pl.BlockSpec((BN, kp), lambda i: (i, 0)),
        out_shape=jax.ShapeDtypeStruct((npad, kp), jnp.float32),
    )(xp, wp)
    return out[:n, :k]


def _prep_edges(src_sorted, seg_sorted):
    e = src_sorted.shape[0]
    srcp = jnp.concatenate([src_sorted, jnp.zeros((EPAD - e,), jnp.int32)])
    segp = jnp.concatenate([seg_sorted, jnp.full((EPAD - e,), N, jnp.int32)])
    qs = jnp.minimum(jnp.arange(129, dtype=jnp.int32) * RS, N)
    b = jnp.searchsorted(segp, qs).astype(jnp.int32)
    bnd = jnp.zeros((144,), jnp.int32).at[:129].set(b)
    return srcp, segp, bnd


def _feast(x, srcp, segp, bnd, W, u, c, b, deg_override=None):
    n = x.shape[0]
    oc = b.shape[0]
    hoc = H * oc
    hocp = _round_up(hoc + 16, 128)
    xWu = _pallas_matmul(x, jnp.concatenate([W, u], axis=1))
    t1 = (jnp.zeros((NPAD, hocp), jnp.float32)
          .at[:n, :hoc].set(xWu[:, :hoc])
          .at[:n, hoc:hoc + H].set(xWu[:, hoc:]))
    xud = jnp.zeros((NPAD, 128), jnp.float32).at[:n, :H].set(xWu[:, hoc:])
    c16 = jnp.zeros((16,), jnp.float32).at[:H].set(c)
    acc = _feast_edge_kernel(hoc, hocp, oc)(t1, xud, srcp, segp, c16, bnd)
    num = acc[:n, :oc]
    deg = acc[:n, oc] if oc < 128 else deg_override
    return num / jnp.maximum(deg, 1.0)[:, None] + b


def _graclus(src, dst, ew, n, mask):
    s = jnp.concatenate([src, dst])
    d = jnp.concatenate([dst, src])
    w = jnp.concatenate([ew, ew])
    m = jnp.concatenate([mask, mask])
    s = jnp.where(m, s, n)
    maxw = jax.ops.segment_max(w, s, num_segments=n + 1)
    cand = jnp.where((w >= maxw[s]) & m, d, n)
    best = jax.ops.segment_min(cand, s, num_segments=n + 1)[:n]
    idx = jnp.arange(n)
    best = jnp.where(best >= n, idx, best)
    mutual = best[best] == idx
    partner = jnp.where(mutual, best, idx)
    return jnp.minimum(idx, partner)


def _relabel(cluster, n):
    present = jnp.zeros(n, jnp.int32).at[cluster].set(1)
    newid = jnp.cumsum(present) - 1
    return newid[cluster]


def _pool_edge(cluster, src, dst, ew, mask, n):
    s = cluster[src]
    d = cluster[dst]
    valid = mask & (s != d)
    big = n * n
    code = jnp.where(valid, d * n + s, big)
    w = jnp.where(valid, ew, 0.0)
    code_s, w_s = jax.lax.sort((code, w), num_keys=1)
    first = jnp.concatenate([jnp.ones((1,), jnp.bool_),
                             code_s[1:] != code_s[:-1]])
    segid = jnp.cumsum(first.astype(jnp.int32)) - 1
    e = code.shape[0]
    nw = jnp.zeros(e, ew.dtype).at[segid].add(w_s)
    ncode = jnp.full(e, big, jnp.int32).at[segid].set(code_s)
    nmask = ncode != big
    nsrc = jnp.where(nmask, ncode % n, 0)
    ndst = jnp.where(nmask, ncode // n, n)
    return nsrc, ndst, nw, nmask


def _pooling_layer(x, src, dst, ew, mask):
    mask = mask & (src != dst)
    clusts = []
    for _ in range(2):
        n = x.shape[0]
        cluster = _graclus(src, dst, ew, n, mask)
        cluster = _relabel(cluster, n)
        clusts.append(cluster)
        x = jax.ops.segment_max(x, cluster, num_segments=n)
        src, dst, ew, mask = _pool_edge(cluster, src, dst, ew, mask, n)
    clust = clusts[-1][clusts[0]]
    return x, src, dst, ew, mask, clust


def kernel(x, edge_index, edge_weight, l1_W, l1_u, l1_c, l1_b, l2_W, l2_u, l2_c, l2_b, l3_W, l3_u, l3_c, l3_b, l4_W, l4_u, l4_c, l4_b, r1_W, r1_u, r1_c, r1_b, r2_W, r2_u, r2_c, r2_b, r3_W, r3_u, r3_c, r3_b, r4_W, r4_u, r4_c, r4_b):
    lr = lambda v: jax.nn.leaky_relu(v, 0.2)
    n = x.shape[0]
    src1, dst1 = edge_index[0], edge_index[1]
    m1 = src1 != dst1
    seg1 = jnp.where(m1, dst1, n)
    sda, ssa = jax.lax.sort((dst1, src1), num_keys=1)
    spa, sga, bna = _prep_edges(ssa, sda)
    sdb, ssb = jax.lax.sort((seg1, src1), num_keys=1)
    spb, sgb, bnb = _prep_edges(ssb, sdb)

    x1 = lr(_feast(x, spa, sga, bna, l1_W, l1_u, l1_c, l1_b))
    x2, src2, dst2, ew2, m2, clust1 = _pooling_layer(
        x1, src1, dst1, edge_weight, jnp.ones(edge_weight.shape, jnp.bool_))
    sp2, sg2, bn2 = _prep_edges(src2, dst2)
    x2 = lr(_feast(x2, sp2, sg2, bn2, l2_W, l2_u, l2_c, l2_b))
    x3, src3, dst3, ew3, m3, clust2 = _pooling_layer(x2, src2, dst2, ew2, m2)
    sp3, sg3, bn3 = _prep_edges(src3, dst3)
    deg3 = jax.ops.segment_sum(jnp.ones(dst3.shape, jnp.float32), dst3,
                               num_segments=n + 1)[:n]
    x3 = lr(_feast(x3, sp3, sg3, bn3, l3_W, l3_u, l3_c, l3_b, deg3))
    x3 = lr(_feast(x3, sp3, sg3, bn3, l4_W, l4_u, l4_c, l4_b, deg3))
    f2 = x3[clust2]
    f2 = _feast(f2, sp2, sg2, bn2, r1_W, r1_u, r1_c, r1_b)
    x2 = jnp.concatenate([x2, f2], axis=1)
    x2 = lr(_feast(x2, sp2, sg2, bn2, r2_W, r2_u, r2_c, r2_b))
    f1 = x2[clust1]
    f1 = _feast(f1, spb, sgb, bnb, r3_W, r3_u, r3_c, r3_b)
    x1 = jnp.concatenate([x1, f1], axis=1)
    out = _feast(x1, spb, sgb, bnb, r4_W, r4_u, r4_c, r4_b)
    return out


# speedup vs baseline: 1.4994x; 1.4994x over previous
import functools

import jax
import jax.numpy as jnp
from jax import lax
from jax.experimental import pallas as pl
from jax.experimental.pallas import tpu as pltpu
from jax.experimental.pallas import tpu_sc as plsc


N = 10000
H = 9
NW = 32
CHUNK = 64
EPAD = N * 32 + 128
NPAD = 10112
RS = 80
NPAD2 = 128 * RS


def _lane_scalar(v, lane):
    lanes = lax.iota(jnp.int32, 16)
    t = jnp.where(lanes == lane, v, jnp.zeros((16,), v.dtype))
    for s in (8, 4, 2, 1):
        t = t + t[(lanes + s) % 16]
    return t[0]


def _feast_edge_body(hoc, hocp, oc, chunk, t1_hbm, xud_hbm, src_hbm, seg_hbm,
                     c_hbm, bnd_hbm, out_hbm, src_v, seg_v, t1_v, xud_v,
                     c_v, bnd_v, acc_v, sem):
    cid = lax.axis_index("c")
    sid = lax.axis_index("s")
    wid = sid * 2 + cid

    def zrow(r, carry):
        for cb in range(8):
            acc_v[r, pl.ds(cb * 16, 16)] = jnp.zeros((16,), jnp.float32)
        return carry

    lax.fori_loop(0, 4 * RS + 8, zrow, 0)
    pltpu.sync_copy(c_hbm, c_v)
    pltpu.sync_copy(bnd_hbm, bnd_v)
    lanes = lax.iota(jnp.int32, 16)
    neg = jnp.where(lanes < H, 0.0, -1e30)
    one0 = jnp.where(lanes < 1, 1.0, 0.0)

    def bnd_at(idx):
        base = pl.multiple_of((idx // 16) * 16, 16)
        return _lane_scalar(bnd_v[pl.ds(base, 16)], idx - base)

    for k in range(4):
        rid = wid + 32 * k
        lo = bnd_at(rid)
        hi = bnd_at(rid + 1)
        off = rid * RS - k * RS
        albase = pl.multiple_of(lo - jnp.mod(lo, 8), 8)
        ntr = jnp.maximum((hi - albase + chunk - 1) // chunk, 0)

        def chunk_body(ch, carry, albase=albase, lo=lo, hi=hi, off=off):
            base = pl.multiple_of(albase + ch * chunk, 8)
            pltpu.sync_copy(src_hbm.at[pl.ds(base, chunk)], src_v)
            pltpu.sync_copy(seg_hbm.at[pl.ds(base, chunk)], seg_v)
            pltpu.async_copy(xud_hbm.at[seg_v], xud_v, sem).wait()
            pltpu.async_copy(t1_hbm.at[src_v], t1_v, sem).wait()

            def edge_body(i, c2):
                sgv = seg_v[pl.ds(pl.multiple_of((i // 16) * 16, 16), 16)]
                sseg = _lane_scalar(sgv, jnp.mod(i, 16))
                j = base + i
                valid = (j >= lo) & (j < hi)
                lrow = jnp.where(valid, sseg - off, 4 * RS)
                z = (t1_v[i, pl.ds(hoc, 16)] - xud_v[i, pl.ds(0, 16)]
                     + c_v[...] + neg)
                m = z
                for s in (8, 4, 2, 1):
                    m = jnp.maximum(m, m[(lanes + s) % 16])
                e = jnp.exp(z - m)
                t = e
                for s in (8, 4, 2, 1):
                    t = t + t[(lanes + s) % 16]
                q = e / t
                accs = [jnp.zeros((16,), jnp.float32) for _ in range(oc // 16)]
                for h in range(H):
                    qh = q[h]
                    for cb in range(oc // 16):
                        accs[cb] = (accs[cb]
                                    + qh * t1_v[i, pl.ds(h * oc + cb * 16, 16)])
                for cb in range(oc // 16):
                    plsc.addupdate(acc_v.at[lrow, pl.ds(cb * 16, 16)], accs[cb])
                if oc < 128:
                    plsc.addupdate(acc_v.at[lrow, pl.ds(oc, 16)], one0)
                return c2

            lax.fori_loop(0, chunk, edge_body, 0)
            return carry

        lax.fori_loop(0, ntr, chunk_body, 0)

    for k in range(4):
        ofs = pl.multiple_of((wid + 32 * k) * RS, 8)
        pltpu.sync_copy(acc_v.at[pl.ds(k * RS, RS)], out_hbm.at[pl.ds(ofs, RS)])


@functools.lru_cache(maxsize=None)
def _feast_edge_kernel(hoc, hocp, oc):
    chunk = 32 if hocp > 640 else 64
    mesh = plsc.VectorSubcoreMesh(core_axis_name="c", subcore_axis_name="s")
    return pl.kernel(
        functools.partial(_feast_edge_body, hoc, hocp, oc, chunk),
        mesh=mesh,
        out_type=jax.ShapeDtypeStruct((NPAD2, 128), jnp.float32),
        scratch_types=[
            pltpu.VMEM((chunk,), jnp.int32),
            pltpu.VMEM((chunk,), jnp.int32),
            pltpu.VMEM((chunk, hocp), jnp.float32),
            pltpu.VMEM((chunk, 128), jnp.float32),
            pltpu.VMEM((16,), jnp.float32),
            pltpu.VMEM((144,), jnp.int32),
            pltpu.VMEM((4 * RS + 8, 128), jnp.float32),
            pltpu.SemaphoreType.DMA,
        ],
    )


def _round_up(v, m):
    return (v + m - 1) // m * m


TROWS = NPAD // 128
GCH = 256
GTRIPS = 79
EG = NW * GCH * GTRIPS


def _node_rmw(s, lanes):
    row = s // 128
    col = s - row * 128
    sub = pl.multiple_of((col // 16) * 16, 16)
    lane = col - sub
    return row, sub, lane


def _seg_max_body(s_hbm, w_hbm, out_hbm, s_v, w_v, tbl_v, sem):
    cid = lax.axis_index("c")
    sid = lax.axis_index("s")
    wid = sid * 2 + cid
    lanes = lax.iota(jnp.int32, 16)
    ninf = jnp.zeros((16,), jnp.float32) - 3.4e38

    def zrow(r, carry):
        for cb in range(8):
            tbl_v[r, pl.ds(cb * 16, 16)] = ninf
        return carry

    lax.fori_loop(0, TROWS, zrow, 0)

    def chunk_body(ch, carry):
        base = wid * (GCH * GTRIPS) + ch * GCH
        pltpu.sync_copy(s_hbm.at[pl.ds(base, GCH)], s_v)
        pltpu.sync_copy(w_hbm.at[pl.ds(base, GCH)], w_v)

        def grp(ii, c2):
            sv = s_v[pl.ds(pl.multiple_of(ii * 16, 16), 16)]
            wv = w_v[pl.ds(pl.multiple_of(ii * 16, 16), 16)]
            for l in range(16):
                s = sv[l]
                ws = wv[l]
                row, sub, lane = _node_rmw(s, lanes)
                vec = tbl_v[row, pl.ds(sub, 16)]
                tbl_v[row, pl.ds(sub, 16)] = jnp.where(
                    lanes == lane, jnp.maximum(vec, ws), vec)
            return c2

        lax.fori_loop(0, GCH // 16, grp, 0)
        return carry

    lax.fori_loop(0, GTRIPS, chunk_body, 0)
    pltpu.sync_copy(tbl_v, out_hbm.at[wid])


def _seg_argmin_body(s_hbm, d_hbm, w_hbm, mx_hbm, out_hbm, s_v, d_v, w_v,
                     mx_v, best_v, sem):
    cid = lax.axis_index("c")
    sid = lax.axis_index("s")
    wid = sid * 2 + cid
    lanes = lax.iota(jnp.int32, 16)
    binit = jnp.zeros((16,), jnp.int32) + (1 << 30)
    pltpu.sync_copy(mx_hbm, mx_v)

    def zrow(r, carry):
        for cb in range(8):
            best_v[r, pl.ds(cb * 16, 16)] = binit
        return carry

    lax.fori_loop(0, TROWS, zrow, 0)

    def chunk_body(ch, carry):
        base = wid * (GCH * GTRIPS) + ch * GCH
        pltpu.sync_copy(s_hbm.at[pl.ds(base, GCH)], s_v)
        pltpu.sync_copy(d_hbm.at[pl.ds(base, GCH)], d_v)
        pltpu.sync_copy(w_hbm.at[pl.ds(base, GCH)], w_v)

        def grp(ii, c2):
            sv = s_v[pl.ds(pl.multiple_of(ii * 16, 16), 16)]
            dv = d_v[pl.ds(pl.multiple_of(ii * 16, 16), 16)]
            wv = w_v[pl.ds(pl.multiple_of(ii * 16, 16), 16)]
            for l in range(16):
                s = sv[l]
                ds = dv[l]
                ws = wv[l]
                row, sub, lane = _node_rmw(s, lanes)
                mrow = mx_v[row, pl.ds(sub, 16)]
                cand = jnp.where(ws >= mrow, ds, N)
                brow = best_v[row, pl.ds(sub, 16)]
                best_v[row, pl.ds(sub, 16)] = jnp.where(
                    lanes == lane, jnp.minimum(brow, cand), brow)
            return c2

        lax.fori_loop(0, GCH // 16, grp, 0)
        return carry

    lax.fori_loop(0, GTRIPS, chunk_body, 0)
    pltpu.sync_copy(best_v, out_hbm.at[wid])


@functools.lru_cache(maxsize=None)
def _seg_max_kernel():
    mesh = plsc.VectorSubcoreMesh(core_axis_name="c", subcore_axis_name="s")
    return pl.kernel(
        _seg_max_body,
        mesh=mesh,
        out_type=jax.ShapeDtypeStruct((NW, TROWS, 128), jnp.float32),
        scratch_types=[
            pltpu.VMEM((GCH,), jnp.int32),
            pltpu.VMEM((GCH,), jnp.float32),
            pltpu.VMEM((TROWS, 128), jnp.float32),
            pltpu.SemaphoreType.DMA,
        ],
    )


@functools.lru_cache(maxsize=None)
def _seg_argmin_kernel():
    mesh = plsc.VectorSubcoreMesh(core_axis_name="c", subcore_axis_name="s")
    return pl.kernel(
        _seg_argmin_body,
        mesh=mesh,
        out_type=jax.ShapeDtypeStruct((NW, TROWS, 128), jnp.int32),
        scratch_types=[
            pltpu.VMEM((GCH,), jnp.int32),
            pltpu.VMEM((GCH,), jnp.int32),
            pltpu.VMEM((GCH,), jnp.float32),
            pltpu.VMEM((TROWS, 128), jnp.float32),
            pltpu.VMEM((TROWS, 128), jnp.int32),
            pltpu.SemaphoreType.DMA,
        ],
    )


def _mm_body(x_ref, w_ref, o_ref):
    o_ref[...] = jnp.dot(x_ref[...], w_ref[...],
                         preferred_element_type=jnp.float32)


def _pallas_matmul(x, w):
    n, ic = x.shape
    k = w.shape[1]
    BN = 1024
    npad = _round_up(n, BN)
    icp = _round_up(ic, 128)
    kp = _round_up(k, 128)
    xp = jnp.zeros((npad, icp), jnp.float32).at[:n, :ic].set(x)
    wp = jnp.zeros((icp, kp), jnp.float32).at[:ic, :k].set(w)
    out = pl.pallas_call(
        _mm_body,
        grid=(npad // BN,),
        in_specs=[pl.BlockSpec((BN, icp), lambda i: (i, 0)),
                  pl.BlockSpec((icp, kp), lambda i: (0, 0))],
        out_specs=pl.BlockSpec((BN, kp), lambda i: (i, 0)),
        out_shape=jax.ShapeDtypeStruct((npad, kp), jnp.float32),
    )(xp, wp)
    return out[:n, :k]


def _prep_edges(src_sorted, seg_sorted):
    e = src_sorted.shape[0]
    srcp = jnp.concatenate([src_sorted, jnp.zeros((EPAD - e,), jnp.int32)])
    segp = jnp.concatenate([seg_sorted, jnp.full((EPAD - e,), N, jnp.int32)])
    qs = jnp.minimum(jnp.arange(129, dtype=jnp.int32) * RS, N)
    b = jnp.searchsorted(segp, qs).astype(jnp.int32)
    bnd = jnp.zeros((144,), jnp.int32).at[:129].set(b)
    return srcp, segp, bnd


def _feast(x, srcp, segp, bnd, W, u, c, b, deg_override=None):
    n = x.shape[0]
    oc = b.shape[0]
    hoc = H * oc
    hocp = _round_up(hoc + 16, 128)
    xWu = _pallas_matmul(x, jnp.concatenate([W, u], axis=1))
    t1 = (jnp.zeros((NPAD, hocp), jnp.float32)
          .at[:n, :hoc].set(xWu[:, :hoc])
          .at[:n, hoc:hoc + H].set(xWu[:, hoc:]))
    xud = jnp.zeros((NPAD, 128), jnp.float32).at[:n, :H].set(xWu[:, hoc:])
    c16 = jnp.zeros((16,), jnp.float32).at[:H].set(c)
    acc = _feast_edge_kernel(hoc, hocp, oc)(t1, xud, srcp, segp, c16, bnd)
    num = acc[:n, :oc]
    deg = acc[:n, oc] if oc < 128 else deg_override
    return num / jnp.maximum(deg, 1.0)[:, None] + b


def _graclus(src, dst, ew, n, mask):
    s = jnp.concatenate([src, dst])
    d = jnp.concatenate([dst, src])
    w = jnp.concatenate([ew, ew])
    m = jnp.concatenate([mask, mask])
    s = jnp.where(m, s, n)
    e2 = s.shape[0]
    sp = jnp.concatenate([s, jnp.full((EG - e2,), n, jnp.int32)])
    dp = jnp.concatenate([d, jnp.full((EG - e2,), n, jnp.int32)])
    wp = jnp.concatenate([w, jnp.full((EG - e2,), -3.4e38, jnp.float32)])
    mx = _seg_max_kernel()(sp, wp).max(axis=0)
    bt = _seg_argmin_kernel()(sp, dp, wp, mx).min(axis=0).reshape(-1)[:n]
    idx = jnp.arange(n)
    best = jnp.where(bt >= n, idx, bt)
    mutual = best[best] == idx
    partner = jnp.where(mutual, best, idx)
    return jnp.minimum(idx, partner)


def _relabel(cluster, n):
    present = jnp.zeros(n, jnp.int32).at[cluster].set(1)
    newid = jnp.cumsum(present) - 1
    return newid[cluster]


def _pool_edge(cluster, src, dst, ew, mask, n):
    s = cluster[src]
    d = cluster[dst]
    valid = mask & (s != d)
    big = n * n
    code = jnp.where(valid, d * n + s, big)
    w = jnp.where(valid, ew, 0.0)
    code_s, w_s = jax.lax.sort((code, w), num_keys=1)
    first = jnp.concatenate([jnp.ones((1,), jnp.bool_),
                             code_s[1:] != code_s[:-1]])
    segid = jnp.cumsum(first.astype(jnp.int32)) - 1
    e = code.shape[0]
    nw = jnp.zeros(e, ew.dtype).at[segid].add(w_s)
    ncode = jnp.full(e, big, jnp.int32).at[segid].set(code_s)
    nmask = ncode != big
    nsrc = jnp.where(nmask, ncode % n, 0)
    ndst = jnp.where(nmask, ncode // n, n)
    return nsrc, ndst, nw, nmask


def _pooling_layer(x, src, dst, ew, mask):
    mask = mask & (src != dst)
    clusts = []
    for _ in range(2):
        n = x.shape[0]
        cluster = _graclus(src, dst, ew, n, mask)
        cluster = _relabel(cluster, n)
        clusts.append(cluster)
        x = jax.ops.segment_max(x, cluster, num_segments=n)
        src, dst, ew, mask = _pool_edge(cluster, src, dst, ew, mask, n)
    clust = clusts[-1][clusts[0]]
    return x, src, dst, ew, mask, clust


def kernel(x, edge_index, edge_weight, l1_W, l1_u, l1_c, l1_b, l2_W, l2_u, l2_c, l2_b, l3_W, l3_u, l3_c, l3_b, l4_W, l4_u, l4_c, l4_b, r1_W, r1_u, r1_c, r1_b, r2_W, r2_u, r2_c, r2_b, r3_W, r3_u, r3_c, r3_b, r4_W, r4_u, r4_c, r4_b):
    lr = lambda v: jax.nn.leaky_relu(v, 0.2)
    n = x.shape[0]
    src1, dst1 = edge_index[0], edge_index[1]
    m1 = src1 != dst1
    seg1 = jnp.where(m1, dst1, n)
    sda, ssa = jax.lax.sort((dst1, src1), num_keys=1)
    spa, sga, bna = _prep_edges(ssa, sda)
    sdb, ssb = jax.lax.sort((seg1, src1), num_keys=1)
    spb, sgb, bnb = _prep_edges(ssb, sdb)

    x1 = lr(_feast(x, spa, sga, bna, l1_W, l1_u, l1_c, l1_b))
    x2, src2, dst2, ew2, m2, clust1 = _pooling_layer(
        x1, src1, dst1, edge_weight, jnp.ones(edge_weight.shape, jnp.bool_))
    sp2, sg2, bn2 = _prep_edges(src2, dst2)
    x2 = lr(_feast(x2, sp2, sg2, bn2, l2_W, l2_u, l2_c, l2_b))
    x3, src3, dst3, ew3, m3, clust2 = _pooling_layer(x2, src2, dst2, ew2, m2)
    sp3, sg3, bn3 = _prep_edges(src3, dst3)
    deg3 = jax.ops.segment_sum(jnp.ones(dst3.shape, jnp.float32), dst3,
                               num_segments=n + 1)[:n]
    x3 = lr(_feast(x3, sp3, sg3, bn3, l3_W, l3_u, l3_c, l3_b, deg3))
    x3 = lr(_feast(x3, sp3, sg3, bn3, l4_W, l4_u, l4_c, l4_b, deg3))
    f2 = x3[clust2]
    f2 = _feast(f2, sp2, sg2, bn2, r1_W, r1_u, r1_c, r1_b)
    x2 = jnp.concatenate([x2, f2], axis=1)
    x2 = lr(_feast(x2, sp2, sg2, bn2, r2_W, r2_u, r2_c, r2_b))
    f1 = x2[clust1]
    f1 = _feast(f1, spb, sgb, bnb, r3_W, r3_u, r3_c, r3_b)
    x1 = jnp.concatenate([x1, f1], axis=1)
    out = _feast(x1, spb, sgb, bnb, r4_W, r4_u, r4_c, r4_b)
    return out


# speedup vs baseline: 1.9997x; 1.3336x over previous
import functools

import jax
import jax.numpy as jnp
from jax import lax
from jax.experimental import pallas as pl
from jax.experimental.pallas import tpu as pltpu
from jax.experimental.pallas import tpu_sc as plsc


N = 10000
H = 9
NW = 32
CHUNK = 64
EPAD = N * 32 + 128
NPAD = 10112
RS = 80
NPAD2 = 128 * RS


def _lane_scalar(v, lane):
    lanes = lax.iota(jnp.int32, 16)
    t = jnp.where(lanes == lane, v, jnp.zeros((16,), v.dtype))
    for s in (8, 4, 2, 1):
        t = t + t[(lanes + s) % 16]
    return t[0]


def _feast_edge_body(hoc, hocp, oc, chunk, t1_hbm, xud_hbm, src_hbm, seg_hbm,
                     c_hbm, bnd_hbm, out_hbm, src_v, seg_v, t1_v, xud_v,
                     c_v, bnd_v, acc_v, sem):
    cid = lax.axis_index("c")
    sid = lax.axis_index("s")
    wid = sid * 2 + cid

    def zrow(r, carry):
        for cb in range(8):
            acc_v[r, pl.ds(cb * 16, 16)] = jnp.zeros((16,), jnp.float32)
        return carry

    lax.fori_loop(0, 4 * RS + 8, zrow, 0)
    pltpu.sync_copy(c_hbm, c_v)
    pltpu.sync_copy(bnd_hbm, bnd_v)
    lanes = lax.iota(jnp.int32, 16)
    neg = jnp.where(lanes < H, 0.0, -1e30)
    one0 = jnp.where(lanes < 1, 1.0, 0.0)

    def bnd_at(idx):
        base = pl.multiple_of((idx // 16) * 16, 16)
        return _lane_scalar(bnd_v[pl.ds(base, 16)], idx - base)

    for k in range(4):
        rid = wid + 32 * k
        lo = bnd_at(rid)
        hi = bnd_at(rid + 1)
        off = rid * RS - k * RS
        albase = pl.multiple_of(lo - jnp.mod(lo, 8), 8)
        ntr = jnp.maximum((hi - albase + chunk - 1) // chunk, 0)

        def chunk_body(ch, carry, albase=albase, lo=lo, hi=hi, off=off):
            base = pl.multiple_of(albase + ch * chunk, 8)
            pltpu.sync_copy(src_hbm.at[pl.ds(base, chunk)], src_v)
            pltpu.sync_copy(seg_hbm.at[pl.ds(base, chunk)], seg_v)
            pltpu.async_copy(xud_hbm.at[seg_v], xud_v, sem).wait()
            pltpu.async_copy(t1_hbm.at[src_v], t1_v, sem).wait()

            def edge_body(i, c2):
                sgv = seg_v[pl.ds(pl.multiple_of((i // 16) * 16, 16), 16)]
                sseg = _lane_scalar(sgv, jnp.mod(i, 16))
                j = base + i
                valid = (j >= lo) & (j < hi)
                lrow = jnp.where(valid, sseg - off, 4 * RS)
                z = (t1_v[i, pl.ds(hoc, 16)] - xud_v[i, pl.ds(0, 16)]
                     + c_v[...] + neg)
                m = z
                for s in (8, 4, 2, 1):
                    m = jnp.maximum(m, m[(lanes + s) % 16])
                e = jnp.exp(z - m)
                t = e
                for s in (8, 4, 2, 1):
                    t = t + t[(lanes + s) % 16]
                q = e / t
                accs = [jnp.zeros((16,), jnp.float32) for _ in range(oc // 16)]
                for h in range(H):
                    qh = q[h]
                    for cb in range(oc // 16):
                        accs[cb] = (accs[cb]
                                    + qh * t1_v[i, pl.ds(h * oc + cb * 16, 16)])
                for cb in range(oc // 16):
                    plsc.addupdate(acc_v.at[lrow, pl.ds(cb * 16, 16)], accs[cb])
                if oc < 128:
                    plsc.addupdate(acc_v.at[lrow, pl.ds(oc, 16)], one0)
                return c2

            lax.fori_loop(0, chunk, edge_body, 0)
            return carry

        lax.fori_loop(0, ntr, chunk_body, 0)

    for k in range(4):
        ofs = pl.multiple_of((wid + 32 * k) * RS, 8)
        pltpu.sync_copy(acc_v.at[pl.ds(k * RS, RS)], out_hbm.at[pl.ds(ofs, RS)])


@functools.lru_cache(maxsize=None)
def _feast_edge_kernel(hoc, hocp, oc):
    chunk = 32 if hocp > 640 else 64
    mesh = plsc.VectorSubcoreMesh(core_axis_name="c", subcore_axis_name="s")
    return pl.kernel(
        functools.partial(_feast_edge_body, hoc, hocp, oc, chunk),
        mesh=mesh,
        out_type=jax.ShapeDtypeStruct((NPAD2, 128), jnp.float32),
        scratch_types=[
            pltpu.VMEM((chunk,), jnp.int32),
            pltpu.VMEM((chunk,), jnp.int32),
            pltpu.VMEM((chunk, hocp), jnp.float32),
            pltpu.VMEM((chunk, 128), jnp.float32),
            pltpu.VMEM((16,), jnp.float32),
            pltpu.VMEM((144,), jnp.int32),
            pltpu.VMEM((4 * RS + 8, 128), jnp.float32),
            pltpu.SemaphoreType.DMA,
        ],
    )


def _round_up(v, m):
    return (v + m - 1) // m * m


TROWS = NPAD // 128
GCH = 256
GTRIPS = 79
EG = NW * GCH * GTRIPS


def _node_rmw(s, lanes):
    row = s // 128
    col = s - row * 128
    sub = pl.multiple_of((col // 16) * 16, 16)
    lane = col - sub
    return row, sub, lane


def _seg_max_body(s_hbm, w_hbm, out_hbm, s_v, w_v, tbl_v, sem):
    cid = lax.axis_index("c")
    sid = lax.axis_index("s")
    wid = sid * 2 + cid
    lanes = lax.iota(jnp.int32, 16)
    ninf = jnp.zeros((16,), jnp.float32) - 3.4e38

    def zrow(r, carry):
        for cb in range(8):
            tbl_v[r, pl.ds(cb * 16, 16)] = ninf
        return carry

    lax.fori_loop(0, TROWS, zrow, 0)

    def chunk_body(ch, carry):
        base = wid * (GCH * GTRIPS) + ch * GCH
        pltpu.sync_copy(s_hbm.at[pl.ds(base, GCH)], s_v)
        pltpu.sync_copy(w_hbm.at[pl.ds(base, GCH)], w_v)

        def grp(ii, c2):
            sv = s_v[pl.ds(pl.multiple_of(ii * 16, 16), 16)]
            wv = w_v[pl.ds(pl.multiple_of(ii * 16, 16), 16)]
            for l in range(16):
                s = sv[l]
                ws = wv[l]
                row, sub, lane = _node_rmw(s, lanes)
                vec = tbl_v[row, pl.ds(sub, 16)]
                tbl_v[row, pl.ds(sub, 16)] = jnp.where(
                    lanes == lane, jnp.maximum(vec, ws), vec)
            return c2

        lax.fori_loop(0, GCH // 16, grp, 0)
        return carry

    lax.fori_loop(0, GTRIPS, chunk_body, 0)
    pltpu.sync_copy(tbl_v, out_hbm.at[wid])


def _seg_argmin_body(s_hbm, d_hbm, w_hbm, mx_hbm, out_hbm, s_v, d_v, w_v,
                     mx_v, best_v, sem):
    cid = lax.axis_index("c")
    sid = lax.axis_index("s")
    wid = sid * 2 + cid
    lanes = lax.iota(jnp.int32, 16)
    binit = jnp.zeros((16,), jnp.int32) + (1 << 30)
    pltpu.sync_copy(mx_hbm, mx_v)

    def zrow(r, carry):
        for cb in range(8):
            best_v[r, pl.ds(cb * 16, 16)] = binit
        return carry

    lax.fori_loop(0, TROWS, zrow, 0)

    def chunk_body(ch, carry):
        base = wid * (GCH * GTRIPS) + ch * GCH
        pltpu.sync_copy(s_hbm.at[pl.ds(base, GCH)], s_v)
        pltpu.sync_copy(d_hbm.at[pl.ds(base, GCH)], d_v)
        pltpu.sync_copy(w_hbm.at[pl.ds(base, GCH)], w_v)

        def grp(ii, c2):
            sv = s_v[pl.ds(pl.multiple_of(ii * 16, 16), 16)]
            dv = d_v[pl.ds(pl.multiple_of(ii * 16, 16), 16)]
            wv = w_v[pl.ds(pl.multiple_of(ii * 16, 16), 16)]
            for l in range(16):
                s = sv[l]
                ds = dv[l]
                ws = wv[l]
                row, sub, lane = _node_rmw(s, lanes)
                mrow = mx_v[row, pl.ds(sub, 16)]
                cand = jnp.where(ws >= mrow, ds, N)
                brow = best_v[row, pl.ds(sub, 16)]
                best_v[row, pl.ds(sub, 16)] = jnp.where(
                    lanes == lane, jnp.minimum(brow, cand), brow)
            return c2

        lax.fori_loop(0, GCH // 16, grp, 0)
        return carry

    lax.fori_loop(0, GTRIPS, chunk_body, 0)
    pltpu.sync_copy(best_v, out_hbm.at[wid])


@functools.lru_cache(maxsize=None)
def _seg_max_kernel():
    mesh = plsc.VectorSubcoreMesh(core_axis_name="c", subcore_axis_name="s")
    return pl.kernel(
        _seg_max_body,
        mesh=mesh,
        out_type=jax.ShapeDtypeStruct((NW, TROWS, 128), jnp.float32),
        scratch_types=[
            pltpu.VMEM((GCH,), jnp.int32),
            pltpu.VMEM((GCH,), jnp.float32),
            pltpu.VMEM((TROWS, 128), jnp.float32),
            pltpu.SemaphoreType.DMA,
        ],
    )


PCH = 256
PTRIPS = 40
EP2 = NW * PCH * PTRIPS


def _pool_map_body(cl_hbm, src_hbm, dst_hbm, m_hbm, code_hbm,
                   cl_v, s_v, d_v, m_v, co_v, sem):
    cid = lax.axis_index("c")
    sid = lax.axis_index("s")
    wid = sid * 2 + cid
    pltpu.sync_copy(cl_hbm, cl_v)
    big = jnp.zeros((16,), jnp.int32) + N * N

    def chunk_body(ch, carry):
        base = wid * (PCH * PTRIPS) + ch * PCH
        pltpu.sync_copy(src_hbm.at[pl.ds(base, PCH)], s_v)
        pltpu.sync_copy(dst_hbm.at[pl.ds(base, PCH)], d_v)
        pltpu.sync_copy(m_hbm.at[pl.ds(base, PCH)], m_v)

        def grp(ii, c2):
            o = pl.multiple_of(ii * 16, 16)
            sv = s_v[pl.ds(o, 16)]
            dv = d_v[pl.ds(o, 16)]
            mv = m_v[pl.ds(o, 16)]
            lanes = lax.iota(jnp.int32, 16)
            co = big
            for l in range(16):
                srow, ssub, slane = _node_rmw(sv[l], lanes)
                drow, dsub, dlane = _node_rmw(dv[l], lanes)
                cs = _lane_scalar(cl_v[srow, pl.ds(ssub, 16)], slane)
                cd = _lane_scalar(cl_v[drow, pl.ds(dsub, 16)], dlane)
                code = jnp.where(cs != cd, cd * N + cs, N * N)
                co = jnp.where(lanes == l, code, co)
            co_v[pl.ds(o, 16)] = jnp.where(mv != 0, co, big)
            return c2

        lax.fori_loop(0, PCH // 16, grp, 0)
        pltpu.sync_copy(co_v, code_hbm.at[pl.ds(base, PCH)])
        return carry

    lax.fori_loop(0, PTRIPS, chunk_body, 0)


@functools.lru_cache(maxsize=None)
def _pool_map_kernel():
    mesh = plsc.VectorSubcoreMesh(core_axis_name="c", subcore_axis_name="s")
    return pl.kernel(
        _pool_map_body,
        mesh=mesh,
        out_type=jax.ShapeDtypeStruct((EP2,), jnp.int32),
        scratch_types=[
            pltpu.VMEM((TROWS, 128), jnp.int32),
            pltpu.VMEM((PCH,), jnp.int32),
            pltpu.VMEM((PCH,), jnp.int32),
            pltpu.VMEM((PCH,), jnp.int32),
            pltpu.VMEM((PCH,), jnp.int32),
            pltpu.SemaphoreType.DMA,
        ],
    )


@functools.lru_cache(maxsize=None)
def _seg_argmin_kernel():
    mesh = plsc.VectorSubcoreMesh(core_axis_name="c", subcore_axis_name="s")
    return pl.kernel(
        _seg_argmin_body,
        mesh=mesh,
        out_type=jax.ShapeDtypeStruct((NW, TROWS, 128), jnp.int32),
        scratch_types=[
            pltpu.VMEM((GCH,), jnp.int32),
            pltpu.VMEM((GCH,), jnp.int32),
            pltpu.VMEM((GCH,), jnp.float32),
            pltpu.VMEM((TROWS, 128), jnp.float32),
            pltpu.VMEM((TROWS, 128), jnp.int32),
            pltpu.SemaphoreType.DMA,
        ],
    )


def _mm_body(x_ref, w_ref, o_ref):
    o_ref[...] = jnp.dot(x_ref[...], w_ref[...],
                         preferred_element_type=jnp.float32)


def _pallas_matmul(x, w):
    n, ic = x.shape
    k = w.shape[1]
    BN = 1024
    npad = _round_up(n, BN)
    icp = _round_up(ic, 128)
    kp = _round_up(k, 128)
    xp = jnp.zeros((npad, icp), jnp.float32).at[:n, :ic].set(x)
    wp = jnp.zeros((icp, kp), jnp.float32).at[:ic, :k].set(w)
    out = pl.pallas_call(
        _mm_body,
        grid=(npad // BN,),
        in_specs=[pl.BlockSpec((BN, icp), lambda i: (i, 0)),
                  pl.BlockSpec((icp, kp), lambda i: (0, 0))],
        out_specs=pl.BlockSpec((BN, kp), lambda i: (i, 0)),
        out_shape=jax.ShapeDtypeStruct((npad, kp), jnp.float32),
    )(xp, wp)
    return out[:n, :k]


def _prep_edges(src_sorted, seg_sorted):
    e = src_sorted.shape[0]
    srcp = jnp.concatenate([src_sorted, jnp.zeros((EPAD - e,), jnp.int32)])
    segp = jnp.concatenate([seg_sorted, jnp.full((EPAD - e,), N, jnp.int32)])
    qs = jnp.minimum(jnp.arange(129, dtype=jnp.int32) * RS, N)
    b = jnp.searchsorted(segp, qs).astype(jnp.int32)
    bnd = jnp.zeros((144,), jnp.int32).at[:129].set(b)
    return srcp, segp, bnd


def _feast(x, srcp, segp, bnd, W, u, c, b, deg_override=None):
    n = x.shape[0]
    oc = b.shape[0]
    hoc = H * oc
    hocp = _round_up(hoc + 16, 128)
    xWu = _pallas_matmul(x, jnp.concatenate([W, u], axis=1))
    t1 = (jnp.zeros((NPAD, hocp), jnp.float32)
          .at[:n, :hoc].set(xWu[:, :hoc])
          .at[:n, hoc:hoc + H].set(xWu[:, hoc:]))
    xud = jnp.zeros((NPAD, 128), jnp.float32).at[:n, :H].set(xWu[:, hoc:])
    c16 = jnp.zeros((16,), jnp.float32).at[:H].set(c)
    acc = _feast_edge_kernel(hoc, hocp, oc)(t1, xud, srcp, segp, c16, bnd)
    num = acc[:n, :oc]
    deg = acc[:n, oc] if oc < 128 else deg_override
    return num / jnp.maximum(deg, 1.0)[:, None] + b


def _graclus(src, dst, ew, n, mask):
    s = jnp.concatenate([src, dst])
    d = jnp.concatenate([dst, src])
    w = jnp.concatenate([ew, ew])
    m = jnp.concatenate([mask, mask])
    s = jnp.where(m, s, n)
    e2 = s.shape[0]
    sp = jnp.concatenate([s, jnp.full((EG - e2,), n, jnp.int32)])
    dp = jnp.concatenate([d, jnp.full((EG - e2,), n, jnp.int32)])
    wp = jnp.concatenate([w, jnp.full((EG - e2,), -3.4e38, jnp.float32)])
    mx = _seg_max_kernel()(sp, wp).max(axis=0)
    bt = _seg_argmin_kernel()(sp, dp, wp, mx).min(axis=0).reshape(-1)[:n]
    idx = jnp.arange(n)
    best = jnp.where(bt >= n, idx, bt)
    mutual = best[best] == idx
    partner = jnp.where(mutual, best, idx)
    return jnp.minimum(idx, partner)


def _relabel(cluster, n):
    present = jnp.zeros(n, jnp.int32).at[cluster].set(1)
    newid = jnp.cumsum(present) - 1
    return newid[cluster]


def _pool_edge(cluster, src, dst, ew, mask, n):
    e = src.shape[0]
    big = n * n
    clp = jnp.zeros((NPAD,), jnp.int32).at[:n].set(cluster).reshape(TROWS, 128)
    pe = EP2 - e
    code = _pool_map_kernel()(
        clp,
        jnp.concatenate([src, jnp.zeros((pe,), jnp.int32)]),
        jnp.concatenate([dst, jnp.zeros((pe,), jnp.int32)]),
        jnp.concatenate([mask.astype(jnp.int32), jnp.zeros((pe,), jnp.int32)]))
    code = code[:e]
    w = jnp.where(code != big, ew, 0.0)
    code_s, w_s = jax.lax.sort((code, w), num_keys=1)
    first = jnp.concatenate([jnp.ones((1,), jnp.bool_),
                             code_s[1:] != code_s[:-1]])
    segid = jnp.cumsum(first.astype(jnp.int32)) - 1
    e = code.shape[0]
    nw = jnp.zeros(e, ew.dtype).at[segid].add(w_s)
    ncode = jnp.full(e, big, jnp.int32).at[segid].set(code_s)
    nmask = ncode != big
    nsrc = jnp.where(nmask, ncode % n, 0)
    ndst = jnp.where(nmask, ncode // n, n)
    return nsrc, ndst, nw, nmask


def _pooling_layer(x, src, dst, ew, mask):
    mask = mask & (src != dst)
    clusts = []
    for _ in range(2):
        n = x.shape[0]
        cluster = _graclus(src, dst, ew, n, mask)
        cluster = _relabel(cluster, n)
        clusts.append(cluster)
        x = jax.ops.segment_max(x, cluster, num_segments=n)
        src, dst, ew, mask = _pool_edge(cluster, src, dst, ew, mask, n)
    clust = clusts[-1][clusts[0]]
    return x, src, dst, ew, mask, clust


def kernel(x, edge_index, edge_weight, l1_W, l1_u, l1_c, l1_b, l2_W, l2_u, l2_c, l2_b, l3_W, l3_u, l3_c, l3_b, l4_W, l4_u, l4_c, l4_b, r1_W, r1_u, r1_c, r1_b, r2_W, r2_u, r2_c, r2_b, r3_W, r3_u, r3_c, r3_b, r4_W, r4_u, r4_c, r4_b):
    lr = lambda v: jax.nn.leaky_relu(v, 0.2)
    n = x.shape[0]
    src1, dst1 = edge_index[0], edge_index[1]
    m1 = src1 != dst1
    seg1 = jnp.where(m1, dst1, n)
    sda, ssa = jax.lax.sort((dst1, src1), num_keys=1)
    spa, sga, bna = _prep_edges(ssa, sda)
    sdb, ssb = jax.lax.sort((seg1, src1), num_keys=1)
    spb, sgb, bnb = _prep_edges(ssb, sdb)

    x1 = lr(_feast(x, spa, sga, bna, l1_W, l1_u, l1_c, l1_b))
    x2, src2, dst2, ew2, m2, clust1 = _pooling_layer(
        x1, src1, dst1, edge_weight, jnp.ones(edge_weight.shape, jnp.bool_))
    sp2, sg2, bn2 = _prep_edges(src2, dst2)
    x2 = lr(_feast(x2, sp2, sg2, bn2, l2_W, l2_u, l2_c, l2_b))
    x3, src3, dst3, ew3, m3, clust2 = _pooling_layer(x2, src2, dst2, ew2, m2)
    sp3, sg3, bn3 = _prep_edges(src3, dst3)
    deg3 = jax.ops.segment_sum(jnp.ones(dst3.shape, jnp.float32), dst3,
                               num_segments=n + 1)[:n]
    x3 = lr(_feast(x3, sp3, sg3, bn3, l3_W, l3_u, l3_c, l3_b, deg3))
    x3 = lr(_feast(x3, sp3, sg3, bn3, l4_W, l4_u, l4_c, l4_b, deg3))
    f2 = x3[clust2]
    f2 = _feast(f2, sp2, sg2, bn2, r1_W, r1_u, r1_c, r1_b)
    x2 = jnp.concatenate([x2, f2], axis=1)
    x2 = lr(_feast(x2, sp2, sg2, bn2, r2_W, r2_u, r2_c, r2_b))
    f1 = x2[clust1]
    f1 = _feast(f1, spb, sgb, bnb, r3_W, r3_u, r3_c, r3_b)
    x1 = jnp.concatenate([x1, f1], axis=1)
    out = _feast(x1, spb, sgb, bnb, r4_W, r4_u, r4_c, r4_b)
    return out


# speedup vs baseline: 2.2152x; 1.1078x over previous
import functools

import jax
import jax.numpy as jnp
from jax import lax
from jax.experimental import pallas as pl
from jax.experimental.pallas import tpu as pltpu
from jax.experimental.pallas import tpu_sc as plsc


N = 10000
H = 9
NW = 32
CHUNK = 64
EPAD = N * 32 + 128
NPAD = 10112
RS = 80
NPAD2 = 128 * RS


def _lane_scalar(v, lane):
    lanes = lax.iota(jnp.int32, 16)
    t = jnp.where(lanes == lane, v, jnp.zeros((16,), v.dtype))
    for s in (8, 4, 2, 1):
        t = t + t[(lanes + s) % 16]
    return t[0]


def _feast_edge_body(hoc, hocp, oc, chunk, t1_hbm, xud_hbm, src_hbm, seg_hbm,
                     c_hbm, bnd_hbm, out_hbm, src_v, seg_v, t1_v, xud_v,
                     c_v, bnd_v, acc_v, sem, sem2):
    cid = lax.axis_index("c")
    sid = lax.axis_index("s")
    wid = sid * 2 + cid

    def zrow(r, carry):
        for cb in range(8):
            acc_v[r, pl.ds(cb * 16, 16)] = jnp.zeros((16,), jnp.float32)
        return carry

    lax.fori_loop(0, 4 * RS + 8, zrow, 0)
    pltpu.sync_copy(c_hbm, c_v)
    pltpu.sync_copy(bnd_hbm, bnd_v)
    lanes = lax.iota(jnp.int32, 16)
    neg = jnp.where(lanes < H, 0.0, -1e30)
    one0 = jnp.where(lanes < 1, 1.0, 0.0)

    def bnd_at(idx):
        base = pl.multiple_of((idx // 16) * 16, 16)
        return _lane_scalar(bnd_v[pl.ds(base, 16)], idx - base)

    for k in range(4):
        rid = wid + 32 * k
        lo = bnd_at(rid)
        hi = bnd_at(rid + 1)
        off = rid * RS - k * RS
        albase = pl.multiple_of(lo - jnp.mod(lo, 8), 8)
        ntr = jnp.maximum((hi - albase + chunk - 1) // chunk, 0)

        def chunk_body(ch, carry, albase=albase, lo=lo, hi=hi, off=off):
            base = pl.multiple_of(albase + ch * chunk, 8)
            pltpu.sync_copy(src_hbm.at[pl.ds(base, chunk)], src_v)
            pltpu.sync_copy(seg_hbm.at[pl.ds(base, chunk)], seg_v)
            cp1 = pltpu.async_copy(xud_hbm.at[seg_v], xud_v, sem)
            cp2 = pltpu.async_copy(t1_hbm.at[src_v], t1_v, sem2)
            cp1.wait()
            cp2.wait()

            def edge_body(i, c2):
                sgv = seg_v[pl.ds(pl.multiple_of((i // 16) * 16, 16), 16)]
                sseg = _lane_scalar(sgv, jnp.mod(i, 16))
                j = base + i
                valid = (j >= lo) & (j < hi)
                lrow = jnp.where(valid, sseg - off, 4 * RS)
                z = (t1_v[i, pl.ds(hoc, 16)] - xud_v[i, pl.ds(0, 16)]
                     + c_v[...] + neg)
                m = z
                for s in (8, 4, 2, 1):
                    m = jnp.maximum(m, m[(lanes + s) % 16])
                e = jnp.exp(z - m)
                t = e
                for s in (8, 4, 2, 1):
                    t = t + t[(lanes + s) % 16]
                q = e / t
                accs = [jnp.zeros((16,), jnp.float32) for _ in range(oc // 16)]
                for h in range(H):
                    qh = q[h]
                    for cb in range(oc // 16):
                        accs[cb] = (accs[cb]
                                    + qh * t1_v[i, pl.ds(h * oc + cb * 16, 16)])
                for cb in range(oc // 16):
                    plsc.addupdate(acc_v.at[lrow, pl.ds(cb * 16, 16)], accs[cb])
                if oc < 128:
                    plsc.addupdate(acc_v.at[lrow, pl.ds(oc, 16)], one0)
                return c2

            lax.fori_loop(0, chunk, edge_body, 0)
            return carry

        lax.fori_loop(0, ntr, chunk_body, 0)

    for k in range(4):
        ofs = pl.multiple_of((wid + 32 * k) * RS, 8)
        pltpu.sync_copy(acc_v.at[pl.ds(k * RS, RS)], out_hbm.at[pl.ds(ofs, RS)])


@functools.lru_cache(maxsize=None)
def _feast_edge_kernel(hoc, hocp, oc):
    chunk = 32 if hocp > 640 else 64
    mesh = plsc.VectorSubcoreMesh(core_axis_name="c", subcore_axis_name="s")
    return pl.kernel(
        functools.partial(_feast_edge_body, hoc, hocp, oc, chunk),
        mesh=mesh,
        out_type=jax.ShapeDtypeStruct((NPAD2, 128), jnp.float32),
        scratch_types=[
            pltpu.VMEM((chunk,), jnp.int32),
            pltpu.VMEM((chunk,), jnp.int32),
            pltpu.VMEM((chunk, hocp), jnp.float32),
            pltpu.VMEM((chunk, 128), jnp.float32),
            pltpu.VMEM((16,), jnp.float32),
            pltpu.VMEM((144,), jnp.int32),
            pltpu.VMEM((4 * RS + 8, 128), jnp.float32),
            pltpu.SemaphoreType.DMA,
            pltpu.SemaphoreType.DMA,
        ],
    )


def _round_up(v, m):
    return (v + m - 1) // m * m


TROWS = NPAD // 128
GCH = 256
GTRIPS = 79
EG = NW * GCH * GTRIPS


def _node_rmw(s, lanes):
    row = s // 128
    col = s - row * 128
    sub = pl.multiple_of((col // 16) * 16, 16)
    lane = col - sub
    return row, sub, lane


def _seg_max_body(s_hbm, w_hbm, out_hbm, s_v, w_v, tbl_v, sem):
    cid = lax.axis_index("c")
    sid = lax.axis_index("s")
    wid = sid * 2 + cid
    lanes = lax.iota(jnp.int32, 16)
    ninf = jnp.zeros((16,), jnp.float32) - 3.4e38

    def zrow(r, carry):
        for cb in range(8):
            tbl_v[r, pl.ds(cb * 16, 16)] = ninf
        return carry

    lax.fori_loop(0, TROWS, zrow, 0)

    def chunk_body(ch, carry):
        base = wid * (GCH * GTRIPS) + ch * GCH
        pltpu.sync_copy(s_hbm.at[pl.ds(base, GCH)], s_v)
        pltpu.sync_copy(w_hbm.at[pl.ds(base, GCH)], w_v)

        def grp(ii, c2):
            sv = s_v[pl.ds(pl.multiple_of(ii * 16, 16), 16)]
            wv = w_v[pl.ds(pl.multiple_of(ii * 16, 16), 16)]
            for l in range(16):
                s = sv[l]
                ws = wv[l]
                row, sub, lane = _node_rmw(s, lanes)
                vec = tbl_v[row, pl.ds(sub, 16)]
                tbl_v[row, pl.ds(sub, 16)] = jnp.where(
                    lanes == lane, jnp.maximum(vec, ws), vec)
            return c2

        lax.fori_loop(0, GCH // 16, grp, 0)
        return carry

    lax.fori_loop(0, GTRIPS, chunk_body, 0)
    pltpu.sync_copy(tbl_v, out_hbm.at[wid])


def _seg_argmin_body(s_hbm, d_hbm, w_hbm, mx_hbm, out_hbm, s_v, d_v, w_v,
                     mx_v, best_v, sem):
    cid = lax.axis_index("c")
    sid = lax.axis_index("s")
    wid = sid * 2 + cid
    lanes = lax.iota(jnp.int32, 16)
    binit = jnp.zeros((16,), jnp.int32) + (1 << 30)
    pltpu.sync_copy(mx_hbm, mx_v)

    def zrow(r, carry):
        for cb in range(8):
            best_v[r, pl.ds(cb * 16, 16)] = binit
        return carry

    lax.fori_loop(0, TROWS, zrow, 0)

    def chunk_body(ch, carry):
        base = wid * (GCH * GTRIPS) + ch * GCH
        pltpu.sync_copy(s_hbm.at[pl.ds(base, GCH)], s_v)
        pltpu.sync_copy(d_hbm.at[pl.ds(base, GCH)], d_v)
        pltpu.sync_copy(w_hbm.at[pl.ds(base, GCH)], w_v)

        def grp(ii, c2):
            sv = s_v[pl.ds(pl.multiple_of(ii * 16, 16), 16)]
            dv = d_v[pl.ds(pl.multiple_of(ii * 16, 16), 16)]
            wv = w_v[pl.ds(pl.multiple_of(ii * 16, 16), 16)]
            for l in range(16):
                s = sv[l]
                ds = dv[l]
                ws = wv[l]
                row, sub, lane = _node_rmw(s, lanes)
                mrow = mx_v[row, pl.ds(sub, 16)]
                cand = jnp.where(ws >= mrow, ds, N)
                brow = best_v[row, pl.ds(sub, 16)]
                best_v[row, pl.ds(sub, 16)] = jnp.where(
                    lanes == lane, jnp.minimum(brow, cand), brow)
            return c2

        lax.fori_loop(0, GCH // 16, grp, 0)
        return carry

    lax.fori_loop(0, GTRIPS, chunk_body, 0)
    pltpu.sync_copy(best_v, out_hbm.at[wid])


@functools.lru_cache(maxsize=None)
def _seg_max_kernel():
    mesh = plsc.VectorSubcoreMesh(core_axis_name="c", subcore_axis_name="s")
    return pl.kernel(
        _seg_max_body,
        mesh=mesh,
        out_type=jax.ShapeDtypeStruct((NW, TROWS, 128), jnp.float32),
        scratch_types=[
            pltpu.VMEM((GCH,), jnp.int32),
            pltpu.VMEM((GCH,), jnp.float32),
            pltpu.VMEM((TROWS, 128), jnp.float32),
            pltpu.SemaphoreType.DMA,
        ],
    )


PCH = 256
PTRIPS = 40
EP2 = NW * PCH * PTRIPS


def _pool_map_body(cl_hbm, src_hbm, dst_hbm, m_hbm, code_hbm,
                   cl_v, s_v, d_v, m_v, co_v, sem):
    cid = lax.axis_index("c")
    sid = lax.axis_index("s")
    wid = sid * 2 + cid
    pltpu.sync_copy(cl_hbm, cl_v)
    big = jnp.zeros((16,), jnp.int32) + N * N

    def chunk_body(ch, carry):
        base = wid * (PCH * PTRIPS) + ch * PCH
        pltpu.sync_copy(src_hbm.at[pl.ds(base, PCH)], s_v)
        pltpu.sync_copy(dst_hbm.at[pl.ds(base, PCH)], d_v)
        pltpu.sync_copy(m_hbm.at[pl.ds(base, PCH)], m_v)

        def grp(ii, c2):
            o = pl.multiple_of(ii * 16, 16)
            sv = s_v[pl.ds(o, 16)]
            dv = d_v[pl.ds(o, 16)]
            mv = m_v[pl.ds(o, 16)]
            lanes = lax.iota(jnp.int32, 16)
            co = big
            for l in range(16):
                srow, ssub, slane = _node_rmw(sv[l], lanes)
                drow, dsub, dlane = _node_rmw(dv[l], lanes)
                cs = _lane_scalar(cl_v[srow, pl.ds(ssub, 16)], slane)
                cd = _lane_scalar(cl_v[drow, pl.ds(dsub, 16)], dlane)
                code = jnp.where(cs != cd, cd * N + cs, N * N)
                co = jnp.where(lanes == l, code, co)
            co_v[pl.ds(o, 16)] = jnp.where(mv != 0, co, big)
            return c2

        lax.fori_loop(0, PCH // 16, grp, 0)
        pltpu.sync_copy(co_v, code_hbm.at[pl.ds(base, PCH)])
        return carry

    lax.fori_loop(0, PTRIPS, chunk_body, 0)


@functools.lru_cache(maxsize=None)
def _pool_map_kernel():
    mesh = plsc.VectorSubcoreMesh(core_axis_name="c", subcore_axis_name="s")
    return pl.kernel(
        _pool_map_body,
        mesh=mesh,
        out_type=jax.ShapeDtypeStruct((EP2,), jnp.int32),
        scratch_types=[
            pltpu.VMEM((TROWS, 128), jnp.int32),
            pltpu.VMEM((PCH,), jnp.int32),
            pltpu.VMEM((PCH,), jnp.int32),
            pltpu.VMEM((PCH,), jnp.int32),
            pltpu.VMEM((PCH,), jnp.int32),
            pltpu.SemaphoreType.DMA,
        ],
    )


@functools.lru_cache(maxsize=None)
def _seg_argmin_kernel():
    mesh = plsc.VectorSubcoreMesh(core_axis_name="c", subcore_axis_name="s")
    return pl.kernel(
        _seg_argmin_body,
        mesh=mesh,
        out_type=jax.ShapeDtypeStruct((NW, TROWS, 128), jnp.int32),
        scratch_types=[
            pltpu.VMEM((GCH,), jnp.int32),
            pltpu.VMEM((GCH,), jnp.int32),
            pltpu.VMEM((GCH,), jnp.float32),
            pltpu.VMEM((TROWS, 128), jnp.float32),
            pltpu.VMEM((TROWS, 128), jnp.int32),
            pltpu.SemaphoreType.DMA,
        ],
    )


def _mm_body(x_ref, w_ref, o_ref):
    o_ref[...] = jnp.dot(x_ref[...], w_ref[...],
                         preferred_element_type=jnp.float32)


def _pallas_matmul(x, w):
    n, ic = x.shape
    k = w.shape[1]
    BN = 1024
    npad = _round_up(n, BN)
    icp = _round_up(ic, 128)
    kp = _round_up(k, 128)
    xp = jnp.zeros((npad, icp), jnp.float32).at[:n, :ic].set(x)
    wp = jnp.zeros((icp, kp), jnp.float32).at[:ic, :k].set(w)
    out = pl.pallas_call(
        _mm_body,
        grid=(npad // BN,),
        in_specs=[pl.BlockSpec((BN, icp), lambda i: (i, 0)),
                  pl.BlockSpec((icp, kp), lambda i: (0, 0))],
        out_specs=pl.BlockSpec((BN, kp), lambda i: (i, 0)),
        out_shape=jax.ShapeDtypeStruct((npad, kp), jnp.float32),
    )(xp, wp)
    return out[:n, :k]


def _prep_edges(src_sorted, seg_sorted):
    e = src_sorted.shape[0]
    srcp = jnp.concatenate([src_sorted, jnp.zeros((EPAD - e,), jnp.int32)])
    segp = jnp.concatenate([seg_sorted, jnp.full((EPAD - e,), N, jnp.int32)])
    qs = jnp.minimum(jnp.arange(129, dtype=jnp.int32) * RS, N)
    b = jnp.searchsorted(segp, qs).astype(jnp.int32)
    bnd = jnp.zeros((144,), jnp.int32).at[:129].set(b)
    return srcp, segp, bnd


def _feast(x, srcp, segp, bnd, W, u, c, b, deg_override=None):
    n = x.shape[0]
    oc = b.shape[0]
    hoc = H * oc
    hocp = _round_up(hoc + 16, 128)
    xWu = _pallas_matmul(x, jnp.concatenate([W, u], axis=1))
    t1 = (jnp.zeros((NPAD, hocp), jnp.float32)
          .at[:n, :hoc].set(xWu[:, :hoc])
          .at[:n, hoc:hoc + H].set(xWu[:, hoc:]))
    xud = jnp.zeros((NPAD, 128), jnp.float32).at[:n, :H].set(xWu[:, hoc:])
    c16 = jnp.zeros((16,), jnp.float32).at[:H].set(c)
    acc = _feast_edge_kernel(hoc, hocp, oc)(t1, xud, srcp, segp, c16, bnd)
    num = acc[:n, :oc]
    deg = acc[:n, oc] if oc < 128 else deg_override
    return num / jnp.maximum(deg, 1.0)[:, None] + b


def _graclus(src, dst, ew, n, mask):
    s = jnp.concatenate([src, dst])
    d = jnp.concatenate([dst, src])
    w = jnp.concatenate([ew, ew])
    m = jnp.concatenate([mask, mask])
    s = jnp.where(m, s, n)
    e2 = s.shape[0]
    sp = jnp.concatenate([s, jnp.full((EG - e2,), n, jnp.int32)])
    dp = jnp.concatenate([d, jnp.full((EG - e2,), n, jnp.int32)])
    wp = jnp.concatenate([w, jnp.full((EG - e2,), -3.4e38, jnp.float32)])
    mx = _seg_max_kernel()(sp, wp).max(axis=0)
    bt = _seg_argmin_kernel()(sp, dp, wp, mx).min(axis=0).reshape(-1)[:n]
    idx = jnp.arange(n)
    best = jnp.where(bt >= n, idx, bt)
    mutual = best[best] == idx
    partner = jnp.where(mutual, best, idx)
    return jnp.minimum(idx, partner)


def _relabel(cluster, n):
    present = jnp.zeros(n, jnp.int32).at[cluster].set(1)
    newid = jnp.cumsum(present) - 1
    return newid[cluster]


def _pool_edge(cluster, src, dst, ew, mask, n):
    e = src.shape[0]
    big = n * n
    clp = jnp.zeros((NPAD,), jnp.int32).at[:n].set(cluster).reshape(TROWS, 128)
    pe = EP2 - e
    code = _pool_map_kernel()(
        clp,
        jnp.concatenate([src, jnp.zeros((pe,), jnp.int32)]),
        jnp.concatenate([dst, jnp.zeros((pe,), jnp.int32)]),
        jnp.concatenate([mask.astype(jnp.int32), jnp.zeros((pe,), jnp.int32)]))
    code = code[:e]
    w = jnp.where(code != big, ew, 0.0)
    code_s, w_s = jax.lax.sort((code, w), num_keys=1)
    first = jnp.concatenate([jnp.ones((1,), jnp.bool_),
                             code_s[1:] != code_s[:-1]])
    segid = jnp.cumsum(first.astype(jnp.int32)) - 1
    e = code.shape[0]
    nw = jnp.zeros(e, ew.dtype).at[segid].add(w_s)
    ncode = jnp.full(e, big, jnp.int32).at[segid].set(code_s)
    nmask = ncode != big
    nsrc = jnp.where(nmask, ncode % n, 0)
    ndst = jnp.where(nmask, ncode // n, n)
    return nsrc, ndst, nw, nmask


def _pooling_layer(x, src, dst, ew, mask):
    mask = mask & (src != dst)
    clusts = []
    for _ in range(2):
        n = x.shape[0]
        cluster = _graclus(src, dst, ew, n, mask)
        cluster = _relabel(cluster, n)
        clusts.append(cluster)
        x = jax.ops.segment_max(x, cluster, num_segments=n)
        src, dst, ew, mask = _pool_edge(cluster, src, dst, ew, mask, n)
    clust = clusts[-1][clusts[0]]
    return x, src, dst, ew, mask, clust


def kernel(x, edge_index, edge_weight, l1_W, l1_u, l1_c, l1_b, l2_W, l2_u, l2_c, l2_b, l3_W, l3_u, l3_c, l3_b, l4_W, l4_u, l4_c, l4_b, r1_W, r1_u, r1_c, r1_b, r2_W, r2_u, r2_c, r2_b, r3_W, r3_u, r3_c, r3_b, r4_W, r4_u, r4_c, r4_b):
    lr = lambda v: jax.nn.leaky_relu(v, 0.2)
    n = x.shape[0]
    src1, dst1 = edge_index[0], edge_index[1]
    m1 = src1 != dst1
    seg1 = jnp.where(m1, dst1, n)
    sda, ssa = jax.lax.sort((dst1, src1), num_keys=1)
    spa, sga, bna = _prep_edges(ssa, sda)
    sdb, ssb = jax.lax.sort((seg1, src1), num_keys=1)
    spb, sgb, bnb = _prep_edges(ssb, sdb)

    x1 = lr(_feast(x, spa, sga, bna, l1_W, l1_u, l1_c, l1_b))
    x2, src2, dst2, ew2, m2, clust1 = _pooling_layer(
        x1, src1, dst1, edge_weight, jnp.ones(edge_weight.shape, jnp.bool_))
    sp2, sg2, bn2 = _prep_edges(src2, dst2)
    x2 = lr(_feast(x2, sp2, sg2, bn2, l2_W, l2_u, l2_c, l2_b))
    x3, src3, dst3, ew3, m3, clust2 = _pooling_layer(x2, src2, dst2, ew2, m2)
    sp3, sg3, bn3 = _prep_edges(src3, dst3)
    deg3 = jax.ops.segment_sum(jnp.ones(dst3.shape, jnp.float32), dst3,
                               num_segments=n + 1)[:n]
    x3 = lr(_feast(x3, sp3, sg3, bn3, l3_W, l3_u, l3_c, l3_b, deg3))
    x3 = lr(_feast(x3, sp3, sg3, bn3, l4_W, l4_u, l4_c, l4_b, deg3))
    f2 = x3[clust2]
    f2 = _feast(f2, sp2, sg2, bn2, r1_W, r1_u, r1_c, r1_b)
    x2 = jnp.concatenate([x2, f2], axis=1)
    x2 = lr(_feast(x2, sp2, sg2, bn2, r2_W, r2_u, r2_c, r2_b))
    f1 = x2[clust1]
    f1 = _feast(f1, spb, sgb, bnb, r3_W, r3_u, r3_c, r3_b)
    x1 = jnp.concatenate([x1, f1], axis=1)
    out = _feast(x1, spb, sgb, bnb, r4_W, r4_u, r4_c, r4_b)
    return out


# speedup vs baseline: 2.2305x; 1.0069x over previous
import functools

import jax
import jax.numpy as jnp
from jax import lax
from jax.experimental import pallas as pl
from jax.experimental.pallas import tpu as pltpu
from jax.experimental.pallas import tpu_sc as plsc


N = 10000
H = 9
NW = 32
CHUNK = 64
EPAD = N * 32 + 128
NPAD = 10112
RS = 80
NPAD2 = 128 * RS


def _lane_scalar(v, lane):
    lanes = lax.iota(jnp.int32, 16)
    t = jnp.where(lanes == lane, v, jnp.zeros((16,), v.dtype))
    for s in (8, 4, 2, 1):
        t = t + t[(lanes + s) % 16]
    return t[0]


def _feast_edge_body(hoc, hocp, oc, chunk, t1_hbm, xud_hbm, src_hbm, seg_hbm,
                     c_hbm, bnd_hbm, out_hbm, src_v, seg_v, t1_v, xud_v,
                     c_v, bnd_v, acc_v, sem, sem2):
    cid = lax.axis_index("c")
    sid = lax.axis_index("s")
    wid = sid * 2 + cid

    def zrow(r, carry):
        for cb in range(8):
            acc_v[r, pl.ds(cb * 16, 16)] = jnp.zeros((16,), jnp.float32)
        return carry

    lax.fori_loop(0, 4 * RS + 8, zrow, 0)
    pltpu.sync_copy(c_hbm, c_v)
    pltpu.sync_copy(bnd_hbm, bnd_v)
    lanes = lax.iota(jnp.int32, 16)
    neg = jnp.where(lanes < H, 0.0, -1e30)
    one0 = jnp.where(lanes < 1, 1.0, 0.0)

    def bnd_at(idx):
        base = pl.multiple_of((idx // 16) * 16, 16)
        return _lane_scalar(bnd_v[pl.ds(base, 16)], idx - base)

    for k in range(4):
        rid = wid + 32 * k
        lo = bnd_at(rid)
        hi = bnd_at(rid + 1)
        off = rid * RS - k * RS
        albase = pl.multiple_of(lo - jnp.mod(lo, 8), 8)
        ntr = jnp.maximum((hi - albase + chunk - 1) // chunk, 0)

        def chunk_body(ch, carry, albase=albase, lo=lo, hi=hi, off=off):
            base = pl.multiple_of(albase + ch * chunk, 8)
            pltpu.sync_copy(src_hbm.at[pl.ds(base, chunk)], src_v)
            pltpu.sync_copy(seg_hbm.at[pl.ds(base, chunk)], seg_v)
            cp1 = pltpu.async_copy(xud_hbm.at[seg_v], xud_v, sem)
            cp2 = pltpu.async_copy(t1_hbm.at[src_v], t1_v, sem2)
            cp1.wait()
            cp2.wait()

            def edge_body(i, c2):
                sgv = seg_v[pl.ds(pl.multiple_of((i // 16) * 16, 16), 16)]
                sseg = _lane_scalar(sgv, jnp.mod(i, 16))
                j = base + i
                valid = (j >= lo) & (j < hi)
                lrow = jnp.where(valid, sseg - off, 4 * RS)
                z = (t1_v[i, pl.ds(hoc, 16)] - xud_v[i, pl.ds(0, 16)]
                     + c_v[...] + neg)
                m = z
                for s in (8, 4, 2, 1):
                    m = jnp.maximum(m, m[(lanes + s) % 16])
                e = jnp.exp(z - m)
                t = e
                for s in (8, 4, 2, 1):
                    t = t + t[(lanes + s) % 16]
                q = e / t
                accs = [jnp.zeros((16,), jnp.float32) for _ in range(oc // 16)]
                for h in range(H):
                    qh = q[h]
                    for cb in range(oc // 16):
                        accs[cb] = (accs[cb]
                                    + qh * t1_v[i, pl.ds(h * oc + cb * 16, 16)])
                for cb in range(oc // 16):
                    plsc.addupdate(acc_v.at[lrow, pl.ds(cb * 16, 16)], accs[cb])
                if oc < 128:
                    plsc.addupdate(acc_v.at[lrow, pl.ds(oc, 16)], one0)
                return c2

            lax.fori_loop(0, chunk, edge_body, 0)
            return carry

        lax.fori_loop(0, ntr, chunk_body, 0)

    for k in range(4):
        ofs = pl.multiple_of((wid + 32 * k) * RS, 8)
        pltpu.sync_copy(acc_v.at[pl.ds(k * RS, RS)], out_hbm.at[pl.ds(ofs, RS)])


@functools.lru_cache(maxsize=None)
def _feast_edge_kernel(hoc, hocp, oc):
    chunk = 48 if hocp > 640 else 64
    mesh = plsc.VectorSubcoreMesh(core_axis_name="c", subcore_axis_name="s")
    return pl.kernel(
        functools.partial(_feast_edge_body, hoc, hocp, oc, chunk),
        mesh=mesh,
        out_type=jax.ShapeDtypeStruct((NPAD2, 128), jnp.float32),
        scratch_types=[
            pltpu.VMEM((chunk,), jnp.int32),
            pltpu.VMEM((chunk,), jnp.int32),
            pltpu.VMEM((chunk, hocp), jnp.float32),
            pltpu.VMEM((chunk, 128), jnp.float32),
            pltpu.VMEM((16,), jnp.float32),
            pltpu.VMEM((144,), jnp.int32),
            pltpu.VMEM((4 * RS + 8, 128), jnp.float32),
            pltpu.SemaphoreType.DMA,
            pltpu.SemaphoreType.DMA,
        ],
    )


def _round_up(v, m):
    return (v + m - 1) // m * m


TROWS = NPAD // 128
GCH = 256
GTRIPS = 79
EG = NW * GCH * GTRIPS


def _node_rmw(s, lanes):
    row = s // 128
    col = s - row * 128
    sub = pl.multiple_of((col // 16) * 16, 16)
    lane = col - sub
    return row, sub, lane


def _seg_max_body(s_hbm, w_hbm, out_hbm, s_v, w_v, tbl_v, sem):
    cid = lax.axis_index("c")
    sid = lax.axis_index("s")
    wid = sid * 2 + cid
    lanes = lax.iota(jnp.int32, 16)
    ninf = jnp.zeros((16,), jnp.float32) - 3.4e38

    def zrow(r, carry):
        for cb in range(8):
            tbl_v[r, pl.ds(cb * 16, 16)] = ninf
        return carry

    lax.fori_loop(0, TROWS, zrow, 0)

    def chunk_body(ch, carry):
        base = wid * (GCH * GTRIPS) + ch * GCH
        pltpu.sync_copy(s_hbm.at[pl.ds(base, GCH)], s_v)
        pltpu.sync_copy(w_hbm.at[pl.ds(base, GCH)], w_v)

        def grp(ii, c2):
            sv = s_v[pl.ds(pl.multiple_of(ii * 16, 16), 16)]
            wv = w_v[pl.ds(pl.multiple_of(ii * 16, 16), 16)]
            for l in range(16):
                s = sv[l]
                ws = wv[l]
                row, sub, lane = _node_rmw(s, lanes)
                vec = tbl_v[row, pl.ds(sub, 16)]
                tbl_v[row, pl.ds(sub, 16)] = jnp.where(
                    lanes == lane, jnp.maximum(vec, ws), vec)
            return c2

        lax.fori_loop(0, GCH // 16, grp, 0)
        return carry

    lax.fori_loop(0, GTRIPS, chunk_body, 0)
    pltpu.sync_copy(tbl_v, out_hbm.at[wid])


def _seg_argmin_body(s_hbm, d_hbm, w_hbm, mx_hbm, out_hbm, s_v, d_v, w_v,
                     mx_v, best_v, sem):
    cid = lax.axis_index("c")
    sid = lax.axis_index("s")
    wid = sid * 2 + cid
    lanes = lax.iota(jnp.int32, 16)
    binit = jnp.zeros((16,), jnp.int32) + (1 << 30)
    pltpu.sync_copy(mx_hbm, mx_v)

    def zrow(r, carry):
        for cb in range(8):
            best_v[r, pl.ds(cb * 16, 16)] = binit
        return carry

    lax.fori_loop(0, TROWS, zrow, 0)

    def chunk_body(ch, carry):
        base = wid * (GCH * GTRIPS) + ch * GCH
        pltpu.sync_copy(s_hbm.at[pl.ds(base, GCH)], s_v)
        pltpu.sync_copy(d_hbm.at[pl.ds(base, GCH)], d_v)
        pltpu.sync_copy(w_hbm.at[pl.ds(base, GCH)], w_v)

        def grp(ii, c2):
            sv = s_v[pl.ds(pl.multiple_of(ii * 16, 16), 16)]
            dv = d_v[pl.ds(pl.multiple_of(ii * 16, 16), 16)]
            wv = w_v[pl.ds(pl.multiple_of(ii * 16, 16), 16)]
            for l in range(16):
                s = sv[l]
                ds = dv[l]
                ws = wv[l]
                row, sub, lane = _node_rmw(s, lanes)
                mrow = mx_v[row, pl.ds(sub, 16)]
                cand = jnp.where(ws >= mrow, ds, N)
                brow = best_v[row, pl.ds(sub, 16)]
                best_v[row, pl.ds(sub, 16)] = jnp.where(
                    lanes == lane, jnp.minimum(brow, cand), brow)
            return c2

        lax.fori_loop(0, GCH // 16, grp, 0)
        return carry

    lax.fori_loop(0, GTRIPS, chunk_body, 0)
    pltpu.sync_copy(best_v, out_hbm.at[wid])


@functools.lru_cache(maxsize=None)
def _seg_max_kernel():
    mesh = plsc.VectorSubcoreMesh(core_axis_name="c", subcore_axis_name="s")
    return pl.kernel(
        _seg_max_body,
        mesh=mesh,
        out_type=jax.ShapeDtypeStruct((NW, TROWS, 128), jnp.float32),
        scratch_types=[
            pltpu.VMEM((GCH,), jnp.int32),
            pltpu.VMEM((GCH,), jnp.float32),
            pltpu.VMEM((TROWS, 128), jnp.float32),
            pltpu.SemaphoreType.DMA,
        ],
    )


PCH = 256
PTRIPS = 40
EP2 = NW * PCH * PTRIPS


def _pool_map_body(cl_hbm, src_hbm, dst_hbm, m_hbm, code_hbm,
                   cl_v, s_v, d_v, m_v, co_v, sem):
    cid = lax.axis_index("c")
    sid = lax.axis_index("s")
    wid = sid * 2 + cid
    pltpu.sync_copy(cl_hbm, cl_v)
    big = jnp.zeros((16,), jnp.int32) + N * N

    def chunk_body(ch, carry):
        base = wid * (PCH * PTRIPS) + ch * PCH
        pltpu.sync_copy(src_hbm.at[pl.ds(base, PCH)], s_v)
        pltpu.sync_copy(dst_hbm.at[pl.ds(base, PCH)], d_v)
        pltpu.sync_copy(m_hbm.at[pl.ds(base, PCH)], m_v)

        def grp(ii, c2):
            o = pl.multiple_of(ii * 16, 16)
            sv = s_v[pl.ds(o, 16)]
            dv = d_v[pl.ds(o, 16)]
            mv = m_v[pl.ds(o, 16)]
            lanes = lax.iota(jnp.int32, 16)
            co = big
            for l in range(16):
                srow, ssub, slane = _node_rmw(sv[l], lanes)
                drow, dsub, dlane = _node_rmw(dv[l], lanes)
                cs = _lane_scalar(cl_v[srow, pl.ds(ssub, 16)], slane)
                cd = _lane_scalar(cl_v[drow, pl.ds(dsub, 16)], dlane)
                code = jnp.where(cs != cd, cd * N + cs, N * N)
                co = jnp.where(lanes == l, code, co)
            co_v[pl.ds(o, 16)] = jnp.where(mv != 0, co, big)
            return c2

        lax.fori_loop(0, PCH // 16, grp, 0)
        pltpu.sync_copy(co_v, code_hbm.at[pl.ds(base, PCH)])
        return carry

    lax.fori_loop(0, PTRIPS, chunk_body, 0)


@functools.lru_cache(maxsize=None)
def _pool_map_kernel():
    mesh = plsc.VectorSubcoreMesh(core_axis_name="c", subcore_axis_name="s")
    return pl.kernel(
        _pool_map_body,
        mesh=mesh,
        out_type=jax.ShapeDtypeStruct((EP2,), jnp.int32),
        scratch_types=[
            pltpu.VMEM((TROWS, 128), jnp.int32),
            pltpu.VMEM((PCH,), jnp.int32),
            pltpu.VMEM((PCH,), jnp.int32),
            pltpu.VMEM((PCH,), jnp.int32),
            pltpu.VMEM((PCH,), jnp.int32),
            pltpu.SemaphoreType.DMA,
        ],
    )


@functools.lru_cache(maxsize=None)
def _seg_argmin_kernel():
    mesh = plsc.VectorSubcoreMesh(core_axis_name="c", subcore_axis_name="s")
    return pl.kernel(
        _seg_argmin_body,
        mesh=mesh,
        out_type=jax.ShapeDtypeStruct((NW, TROWS, 128), jnp.int32),
        scratch_types=[
            pltpu.VMEM((GCH,), jnp.int32),
            pltpu.VMEM((GCH,), jnp.int32),
            pltpu.VMEM((GCH,), jnp.float32),
            pltpu.VMEM((TROWS, 128), jnp.float32),
            pltpu.VMEM((TROWS, 128), jnp.int32),
            pltpu.SemaphoreType.DMA,
        ],
    )


def _mm_body(x_ref, w_ref, o_ref):
    o_ref[...] = jnp.dot(x_ref[...], w_ref[...],
                         preferred_element_type=jnp.float32)


def _pallas_matmul(x, w):
    n, ic = x.shape
    k = w.shape[1]
    BN = 1024
    npad = _round_up(n, BN)
    icp = _round_up(ic, 128)
    kp = _round_up(k, 128)
    xp = jnp.zeros((npad, icp), jnp.float32).at[:n, :ic].set(x)
    wp = jnp.zeros((icp, kp), jnp.float32).at[:ic, :k].set(w)
    out = pl.pallas_call(
        _mm_body,
        grid=(npad // BN,),
        in_specs=[pl.BlockSpec((BN, icp), lambda i: (i, 0)),
                  pl.BlockSpec((icp, kp), lambda i: (0, 0))],
        out_specs=pl.BlockSpec((BN, kp), lambda i: (i, 0)),
        out_shape=jax.ShapeDtypeStruct((npad, kp), jnp.float32),
    )(xp, wp)
    return out[:n, :k]


def _prep_edges(src_sorted, seg_sorted):
    e = src_sorted.shape[0]
    srcp = jnp.concatenate([src_sorted, jnp.zeros((EPAD - e,), jnp.int32)])
    segp = jnp.concatenate([seg_sorted, jnp.full((EPAD - e,), N, jnp.int32)])
    qs = jnp.minimum(jnp.arange(129, dtype=jnp.int32) * RS, N)
    b = jnp.searchsorted(segp, qs).astype(jnp.int32)
    bnd = jnp.zeros((144,), jnp.int32).at[:129].set(b)
    return srcp, segp, bnd


def _feast(x, srcp, segp, bnd, W, u, c, b, deg_override=None):
    n = x.shape[0]
    oc = b.shape[0]
    hoc = H * oc
    hocp = _round_up(hoc + 16, 128)
    xWu = _pallas_matmul(x, jnp.concatenate([W, u], axis=1))
    t1 = (jnp.zeros((NPAD, hocp), jnp.float32)
          .at[:n, :hoc].set(xWu[:, :hoc])
          .at[:n, hoc:hoc + H].set(xWu[:, hoc:]))
    xud = jnp.zeros((NPAD, 128), jnp.float32).at[:n, :H].set(xWu[:, hoc:])
    c16 = jnp.zeros((16,), jnp.float32).at[:H].set(c)
    acc = _feast_edge_kernel(hoc, hocp, oc)(t1, xud, srcp, segp, c16, bnd)
    num = acc[:n, :oc]
    deg = acc[:n, oc] if oc < 128 else deg_override
    return num / jnp.maximum(deg, 1.0)[:, None] + b


def _graclus(src, dst, ew, n, mask):
    s = jnp.concatenate([src, dst])
    d = jnp.concatenate([dst, src])
    w = jnp.concatenate([ew, ew])
    m = jnp.concatenate([mask, mask])
    s = jnp.where(m, s, n)
    e2 = s.shape[0]
    sp = jnp.concatenate([s, jnp.full((EG - e2,), n, jnp.int32)])
    dp = jnp.concatenate([d, jnp.full((EG - e2,), n, jnp.int32)])
    wp = jnp.concatenate([w, jnp.full((EG - e2,), -3.4e38, jnp.float32)])
    mx = _seg_max_kernel()(sp, wp).max(axis=0)
    bt = _seg_argmin_kernel()(sp, dp, wp, mx).min(axis=0).reshape(-1)[:n]
    idx = jnp.arange(n)
    best = jnp.where(bt >= n, idx, bt)
    mutual = best[best] == idx
    partner = jnp.where(mutual, best, idx)
    return jnp.minimum(idx, partner)


def _relabel(cluster, n):
    present = jnp.zeros(n, jnp.int32).at[cluster].set(1)
    newid = jnp.cumsum(present) - 1
    return newid[cluster]


def _pool_edge(cluster, src, dst, ew, mask, n):
    e = src.shape[0]
    big = n * n
    clp = jnp.zeros((NPAD,), jnp.int32).at[:n].set(cluster).reshape(TROWS, 128)
    pe = EP2 - e
    code = _pool_map_kernel()(
        clp,
        jnp.concatenate([src, jnp.zeros((pe,), jnp.int32)]),
        jnp.concatenate([dst, jnp.zeros((pe,), jnp.int32)]),
        jnp.concatenate([mask.astype(jnp.int32), jnp.zeros((pe,), jnp.int32)]))
    code = code[:e]
    w = jnp.where(code != big, ew, 0.0)
    code_s, w_s = jax.lax.sort((code, w), num_keys=1)
    first = jnp.concatenate([jnp.ones((1,), jnp.bool_),
                             code_s[1:] != code_s[:-1]])
    segid = jnp.cumsum(first.astype(jnp.int32)) - 1
    e = code.shape[0]
    nw = jnp.zeros(e, ew.dtype).at[segid].add(w_s)
    ncode = jnp.full(e, big, jnp.int32).at[segid].set(code_s)
    nmask = ncode != big
    nsrc = jnp.where(nmask, ncode % n, 0)
    ndst = jnp.where(nmask, ncode // n, n)
    return nsrc, ndst, nw, nmask


def _pooling_layer(x, src, dst, ew, mask):
    mask = mask & (src != dst)
    clusts = []
    for _ in range(2):
        n = x.shape[0]
        cluster = _graclus(src, dst, ew, n, mask)
        cluster = _relabel(cluster, n)
        clusts.append(cluster)
        x = jax.ops.segment_max(x, cluster, num_segments=n)
        src, dst, ew, mask = _pool_edge(cluster, src, dst, ew, mask, n)
    clust = clusts[-1][clusts[0]]
    return x, src, dst, ew, mask, clust


def kernel(x, edge_index, edge_weight, l1_W, l1_u, l1_c, l1_b, l2_W, l2_u, l2_c, l2_b, l3_W, l3_u, l3_c, l3_b, l4_W, l4_u, l4_c, l4_b, r1_W, r1_u, r1_c, r1_b, r2_W, r2_u, r2_c, r2_b, r3_W, r3_u, r3_c, r3_b, r4_W, r4_u, r4_c, r4_b):
    lr = lambda v: jax.nn.leaky_relu(v, 0.2)
    n = x.shape[0]
    src1, dst1 = edge_index[0], edge_index[1]
    m1 = src1 != dst1
    seg1 = jnp.where(m1, dst1, n)
    sda, ssa = jax.lax.sort((dst1, src1), num_keys=1)
    spa, sga, bna = _prep_edges(ssa, sda)
    sdb, ssb = jax.lax.sort((seg1, src1), num_keys=1)
    spb, sgb, bnb = _prep_edges(ssb, sdb)

    x1 = lr(_feast(x, spa, sga, bna, l1_W, l1_u, l1_c, l1_b))
    x2, src2, dst2, ew2, m2, clust1 = _pooling_layer(
        x1, src1, dst1, edge_weight, jnp.ones(edge_weight.shape, jnp.bool_))
    sp2, sg2, bn2 = _prep_edges(src2, dst2)
    x2 = lr(_feast(x2, sp2, sg2, bn2, l2_W, l2_u, l2_c, l2_b))
    x3, src3, dst3, ew3, m3, clust2 = _pooling_layer(x2, src2, dst2, ew2, m2)
    sp3, sg3, bn3 = _prep_edges(src3, dst3)
    deg3 = jax.ops.segment_sum(jnp.ones(dst3.shape, jnp.float32), dst3,
                               num_segments=n + 1)[:n]
    x3 = lr(_feast(x3, sp3, sg3, bn3, l3_W, l3_u, l3_c, l3_b, deg3))
    x3 = lr(_feast(x3, sp3, sg3, bn3, l4_W, l4_u, l4_c, l4_b, deg3))
    f2 = x3[clust2]
    f2 = _feast(f2, sp2, sg2, bn2, r1_W, r1_u, r1_c, r1_b)
    x2 = jnp.concatenate([x2, f2], axis=1)
    x2 = lr(_feast(x2, sp2, sg2, bn2, r2_W, r2_u, r2_c, r2_b))
    f1 = x2[clust1]
    f1 = _feast(f1, spb, sgb, bnb, r3_W, r3_u, r3_c, r3_b)
    x1 = jnp.concatenate([x1, f1], axis=1)
    out = _feast(x1, spb, sgb, bnb, r4_W, r4_u, r4_c, r4_b)
    return out
